# speedup vs baseline: 1.0055x; 1.0055x over previous
import jax
import jax.numpy as jnp
from jax.experimental import pallas as pl


def _ln_body(x_ref, g_ref, b_ref, o_ref):
    x = x_ref[...]
    mu = jnp.mean(x, axis=-1, keepdims=True)
    var = jnp.mean((x - mu) ** 2, axis=-1, keepdims=True)
    o_ref[...] = (x - mu) * jax.lax.rsqrt(var + 1e-5) * g_ref[...] + b_ref[...]


def _ln(x, g, b):
    n, f = x.shape
    blk = 1000
    return pl.pallas_call(
        _ln_body,
        grid=(n // blk,),
        in_specs=[
            pl.BlockSpec((blk, f), lambda i: (i, 0)),
            pl.BlockSpec((f,), lambda i: (0,)),
            pl.BlockSpec((f,), lambda i: (0,)),
        ],
        out_specs=pl.BlockSpec((blk, f), lambda i: (i, 0)),
        out_shape=jax.ShapeDtypeStruct((n, f), x.dtype),
    )(x, g, b)


def _gatv2(x, src, dst, Wl, bl, Wr, br, att, bias, heads, outc):
    n = x.shape[0]
    xl = (x @ Wl.T + bl).reshape(n, heads, outc)
    xr = (x @ Wr.T + br).reshape(n, heads, outc)
    m = xl[src] + xr[dst]
    m = jnp.where(m > 0, m, 0.2 * m)
    e = jnp.sum(m * att[None, :, :], axis=-1)
    emax = jax.ops.segment_max(e, dst, num_segments=n)
    emax = jnp.where(jnp.isfinite(emax), emax, 0.0)
    ex = jnp.exp(e - emax[dst])
    den = jax.ops.segment_sum(ex, dst, num_segments=n)
    alpha = ex / (den[dst] + 1e-16)
    out = jax.ops.segment_sum(xl[src] * alpha[:, :, None], dst, num_segments=n)
    return out.reshape(n, heads * outc) + bias


def kernel(x, edge_index, Wl1, bl1, Wr1, br1, att1, bias1, g1, beta1,
           Wl2, bl2, Wr2, br2, att2, bias2, g2, beta2):
    n = x.shape[0]
    loop = jnp.arange(n, dtype=edge_index.dtype)
    src = jnp.concatenate([edge_index[0], loop])
    dst = jnp.concatenate([edge_index[1], loop])
    h = _gatv2(x, src, dst, Wl1, bl1, Wr1, br1, att1, bias1, 4, 16)
    h = _ln(h, g1, beta1)
    h = jax.nn.elu(h)
    h = _gatv2(h, src, dst, Wl2, bl2, Wr2, br2, att2, bias2, 1, 32)
    h = _ln(h, g2, beta2)
    return h


# speedup vs baseline: 4.8602x; 4.8336x over previous
import functools

import jax
import jax.numpy as jnp
from jax import lax
from jax.experimental import pallas as pl
from jax.experimental.pallas import tpu as pltpu
from jax.experimental.pallas import tpu_sc as plsc

N_NODES = 100000
NCH = 6
CH = 17000
CHP = 17024
RPT = CHP // 16
KB = 128
NB = 832
PER_TILE = KB * NB
E_PAD = 16 * PER_TILE


def _make_edge_kernel(heads, outc):
    d = heads * outc
    w = d + 16
    nv = d // 16
    mesh = plsc.VectorSubcoreMesh(core_axis_name="c", subcore_axis_name="s")

    @functools.partial(
        pl.kernel,
        mesh=mesh,
        compiler_params=pltpu.CompilerParams(
            needs_layout_passes=False, use_tc_tiling_on_sc=False),
        out_type=jax.ShapeDtypeStruct((NCH, CHP, w), jnp.float32),
        scratch_types=[
            pltpu.VMEM((KB,), jnp.int32),
            pltpu.VMEM((KB,), jnp.int32),
            pltpu.VMEM((KB,), jnp.int32),
            pltpu.VMEM((KB,), jnp.int32),
            pltpu.VMEM((KB,), jnp.float32),
            pltpu.VMEM((KB, d), jnp.float32),
            pltpu.VMEM((KB, d), jnp.float32),
            pltpu.VMEM((KB, w), jnp.float32),
            pltpu.VMEM((d,), jnp.float32),
            pltpu.VMEM_SHARED((CHP, w), jnp.float32),
            pltpu.SemaphoreType.DMA,
            pltpu.SemaphoreType.DMA,
        ],
    )
    def edge_kernel(src_hbm, dst_hbm, xl_hbm, xr_hbm, att_hbm, acc_hbm,
                    srcv, dstv, gdstv, dlocv, wtv, xlv, xrv, valsv, attv,
                    accsh, sem1, sem2):
        cid = lax.axis_index("c")
        sid = lax.axis_index("s")
        pltpu.sync_copy(att_hbm, attv)
        att_s = None
        zero16 = jnp.zeros((16,), jnp.float32)
        lanes = lax.iota(jnp.int32, 16)

        for rnd in range(NCH // 2):
            chunk = cid * (NCH // 2) + rnd
            lo = chunk * CH

            def zrow(j, carry):
                for v in range(w // 16):
                    valsv[j, pl.ds(v * 16, 16)] = zero16
                return carry

            lax.fori_loop(0, KB, zrow, 0)
            base = sid * RPT
            nfull = RPT // KB
            rem = RPT - nfull * KB
            for j in range(nfull):
                pltpu.sync_copy(valsv, accsh.at[pl.ds(base + j * KB, KB)])
            pltpu.sync_copy(valsv.at[pl.ds(0, rem)],
                            accsh.at[pl.ds(base + nfull * KB, rem)])
            plsc.subcore_barrier()

            def batch(bi, carry):
                off = sid * PER_TILE + bi * KB
                pltpu.sync_copy(src_hbm.at[pl.ds(off, KB)], srcv)
                pltpu.sync_copy(dst_hbm.at[pl.ds(off, KB)], dstv)

                def mask16(j, c2):
                    dv = dstv[pl.ds(j * 16, 16)]
                    inm = (dv >= lo) & (dv < lo + CH)
                    dlocv[pl.ds(j * 16, 16)] = jnp.where(inm, dv - lo, 0)
                    wtv[pl.ds(j * 16, 16)] = jnp.where(inm, 1.0, 0.0)
                    gdstv[pl.ds(j * 16, 16)] = jnp.minimum(dv, N_NODES - 1)
                    return c2

                lax.fori_loop(0, KB // 16, mask16, 0)
                cp1 = pltpu.async_copy(xl_hbm.at[srcv], xlv, sem1)
                cp2 = pltpu.async_copy(xr_hbm.at[gdstv], xrv, sem2)
                cp1.wait()
                cp2.wait()

                def egroup(g, c3):
                    rows = lanes + g * 16
                    wt16 = wtv[pl.ds(g * 16, 16)]
                    for h in range(heads):
                        acc = zero16
                        for c in range(outc):
                            o = h * outc + c
                            colv = jnp.full((16,), o, jnp.int32)
                            a = plsc.load_gather(xlv, [rows, colv])
                            b = plsc.load_gather(xrv, [rows, colv])
                            m = a + b
                            m = jnp.where(m > 0, m, m * 0.2)
                            acc = acc + m * att_s[o]
                        ex = jnp.exp(acc) * wt16
                        for c in range(outc):
                            o = h * outc + c
                            colv = jnp.full((16,), o, jnp.int32)
                            a = plsc.load_gather(xlv, [rows, colv])
                            plsc.store_scatter(valsv, [rows, colv], a * ex)
                        plsc.store_scatter(
                            valsv, [rows, jnp.full((16,), d + h, jnp.int32)],
                            ex)
                    return c3

                lax.fori_loop(0, KB // 16, egroup, 0)
                pltpu.sync_copy(valsv, accsh.at[dlocv], add=True)
                return carry

            if att_s is None:
                att_vregs = [attv[pl.ds(v * 16, 16)] for v in range(d // 16)]
                att_s = [att_vregs[o // 16][o % 16] for o in range(d)]
            lax.fori_loop(0, NB, batch, 0)
            plsc.subcore_barrier()
            pltpu.sync_copy(accsh.at[pl.ds(base, RPT)],
                            acc_hbm.at[chunk, pl.ds(base, RPT)])
            plsc.subcore_barrier()

    return edge_kernel


_edge1 = _make_edge_kernel(4, 16)
_edge2 = _make_edge_kernel(1, 32)


def _proj1_body(x_ref, wl_ref, bl_ref, wr_ref, br_ref, xl_ref, xr_ref):
    xb = x_ref[...]
    dn = (((1,), (1,)), ((), ()))
    xl_ref[...] = lax.dot_general(
        xb, wl_ref[...], dn, preferred_element_type=jnp.float32) + bl_ref[...]
    xr_ref[...] = lax.dot_general(
        xb, wr_ref[...], dn, preferred_element_type=jnp.float32) + br_ref[...]


def _proj1(x, wl, bl, wr, br):
    n = x.shape[0]
    blk = 1000
    return pl.pallas_call(
        _proj1_body,
        grid=(n // blk,),
        in_specs=[
            pl.BlockSpec((blk, 5), lambda i: (i, 0)),
            pl.BlockSpec((64, 5), lambda i: (0, 0)),
            pl.BlockSpec((64,), lambda i: (0,)),
            pl.BlockSpec((64, 5), lambda i: (0, 0)),
            pl.BlockSpec((64,), lambda i: (0,)),
        ],
        out_specs=[
            pl.BlockSpec((blk, 64), lambda i: (i, 0)),
            pl.BlockSpec((blk, 64), lambda i: (i, 0)),
        ],
        out_shape=[
            jax.ShapeDtypeStruct((n, 64), jnp.float32),
            jax.ShapeDtypeStruct((n, 64), jnp.float32),
        ],
    )(x, wl, bl, wr, br)


def _finish1_body(acc_ref, b1_ref, g1_ref, be1_ref, wl2_ref, bl2_ref,
                  wr2_ref, br2_ref, xl2_ref, xr2_ref):
    blk = acc_ref[0]
    num = blk[:, :64]
    den4 = blk[:, 64:68]
    denf = jnp.reshape(
        jnp.broadcast_to(den4[:, :, None], (den4.shape[0], 4, 16)),
        (den4.shape[0], 64))
    h = num / (denf + 1e-16) + b1_ref[...]
    mu = jnp.mean(h, axis=-1, keepdims=True)
    var = jnp.mean((h - mu) ** 2, axis=-1, keepdims=True)
    h = (h - mu) * lax.rsqrt(var + 1e-5) * g1_ref[...] + be1_ref[...]
    h = jnp.where(h > 0, h, jnp.exp(h) - 1.0)
    dn = (((1,), (1,)), ((), ()))
    xl2_ref[...] = lax.dot_general(
        h, wl2_ref[...], dn, preferred_element_type=jnp.float32) + bl2_ref[...]
    xr2_ref[...] = lax.dot_general(
        h, wr2_ref[...], dn, preferred_element_type=jnp.float32) + br2_ref[...]


def _finish1(acc, b1, g1, be1, wl2, bl2, wr2, br2):
    blk = 1000
    nb = CH // blk
    return pl.pallas_call(
        _finish1_body,
        grid=(NCH, nb),
        in_specs=[
            pl.BlockSpec((1, blk, 80), lambda c, r: (c, r, 0)),
            pl.BlockSpec((64,), lambda c, r: (0,)),
            pl.BlockSpec((64,), lambda c, r: (0,)),
            pl.BlockSpec((64,), lambda c, r: (0,)),
            pl.BlockSpec((32, 64), lambda c, r: (0, 0)),
            pl.BlockSpec((32,), lambda c, r: (0,)),
            pl.BlockSpec((32, 64), lambda c, r: (0, 0)),
            pl.BlockSpec((32,), lambda c, r: (0,)),
        ],
        out_specs=[
            pl.BlockSpec((blk, 32), lambda c, r: (c * nb + r, 0)),
            pl.BlockSpec((blk, 32), lambda c, r: (c * nb + r, 0)),
        ],
        out_shape=[
            jax.ShapeDtypeStruct((NCH * CH, 32), jnp.float32),
            jax.ShapeDtypeStruct((NCH * CH, 32), jnp.float32),
        ],
    )(acc, b1, g1, be1, wl2, bl2, wr2, br2)


def _finish2_body(acc_ref, b2_ref, g2_ref, be2_ref, o_ref):
    blk = acc_ref[0]
    num = blk[:, :32]
    den = blk[:, 32:33]
    h = num / (jnp.broadcast_to(den, num.shape) + 1e-16) + b2_ref[...]
    mu = jnp.mean(h, axis=-1, keepdims=True)
    var = jnp.mean((h - mu) ** 2, axis=-1, keepdims=True)
    o_ref[...] = (h - mu) * lax.rsqrt(var + 1e-5) * g2_ref[...] + be2_ref[...]


def _finish2(acc, b2, g2, be2):
    blk = 1000
    nb = CH // blk
    return pl.pallas_call(
        _finish2_body,
        grid=(NCH, nb),
        in_specs=[
            pl.BlockSpec((1, blk, 48), lambda c, r: (c, r, 0)),
            pl.BlockSpec((32,), lambda c, r: (0,)),
            pl.BlockSpec((32,), lambda c, r: (0,)),
            pl.BlockSpec((32,), lambda c, r: (0,)),
        ],
        out_specs=pl.BlockSpec((blk, 32), lambda c, r: (c * nb + r, 0)),
        out_shape=jax.ShapeDtypeStruct((NCH * CH, 32), jnp.float32),
    )(acc, b2, g2, be2)


def kernel(x, edge_index, Wl1, bl1, Wr1, br1, att1, bias1, g1, beta1,
           Wl2, bl2, Wr2, br2, att2, bias2, g2, beta2):
    n = x.shape[0]
    e = edge_index.shape[1]
    loop = jnp.arange(n, dtype=edge_index.dtype)
    pad = E_PAD - (e + n)
    src = jnp.concatenate(
        [edge_index[0], loop, jnp.zeros((pad,), edge_index.dtype)])
    dst = jnp.concatenate(
        [edge_index[1], loop,
         jnp.full((pad,), jnp.int32(1 << 30), edge_index.dtype)])

    xl1, xr1 = _proj1(x, Wl1, bl1, Wr1, br1)
    acc1 = _edge1(src, dst, xl1, xr1, att1.reshape(64))
    xl2, xr2 = _finish1(acc1, bias1, g1, beta1, Wl2, bl2, Wr2, br2)
    acc2 = _edge2(src, dst, xl2, xr2, att2.reshape(32))
    return _finish2(acc2, bias2, g2, beta2)[:n]


# speedup vs baseline: 8.1930x; 1.6857x over previous
import functools

import jax
import jax.numpy as jnp
from jax import lax
from jax.experimental import pallas as pl
from jax.experimental.pallas import tpu as pltpu
from jax.experimental.pallas import tpu_sc as plsc

N_NODES = 100000
NCH = 6
CH = 17000
CHP = 17024
RPT = CHP // 16
KB = 128
NB = 832
PER_TILE = KB * NB
E_PAD = 16 * PER_TILE
E_REAL = 1700000


def _make_edge_kernel(heads, outc):
    d = heads * outc
    w = d + 16
    nv = d // 16
    mesh = plsc.VectorSubcoreMesh(core_axis_name="c", subcore_axis_name="s")

    @functools.partial(
        pl.kernel,
        mesh=mesh,
        compiler_params=pltpu.CompilerParams(
            needs_layout_passes=False, use_tc_tiling_on_sc=False),
        out_type=jax.ShapeDtypeStruct((NCH, CHP, w), jnp.float32),
        scratch_types=[
            pltpu.VMEM((2, KB), jnp.int32),
            pltpu.VMEM((2, KB), jnp.int32),
            pltpu.VMEM((KB,), jnp.int32),
            pltpu.VMEM((KB,), jnp.float32),
            pltpu.VMEM((2, KB, d), jnp.float32),
            pltpu.VMEM((2, KB, d), jnp.float32),
            pltpu.VMEM((KB, w), jnp.float32),
            pltpu.VMEM((d,), jnp.float32),
            pltpu.VMEM_SHARED((CHP, w), jnp.float32),
            pltpu.SemaphoreType.DMA,
            pltpu.SemaphoreType.DMA,
        ],
    )
    def edge_kernel(src_hbm, dst_hbm, xl_hbm, xr_hbm, att_hbm, acc_hbm,
                    srcv, dstv, dlocv, wtv, xlv, xrv, valsv, attv,
                    accsh, semg0, semg1):
        cid = lax.axis_index("c")
        sid = lax.axis_index("s")
        semg = (semg0, semg1)
        pltpu.sync_copy(att_hbm, attv)
        att_s = None
        zero16 = jnp.zeros((16,), jnp.float32)
        lanes = lax.iota(jnp.int32, 16)

        for rnd in range(NCH // 2):
            chunk = cid * (NCH // 2) + rnd
            lo = chunk * CH

            def zrow(j, carry):
                for v in range(w // 16):
                    valsv[j, pl.ds(v * 16, 16)] = zero16
                return carry

            lax.fori_loop(0, KB, zrow, 0)
            base = sid * RPT
            nfull = RPT // KB
            rem = RPT - nfull * KB
            for j in range(nfull):
                pltpu.sync_copy(valsv, accsh.at[pl.ds(base + j * KB, KB)])
            pltpu.sync_copy(valsv.at[pl.ds(0, rem)],
                            accsh.at[pl.ds(base + nfull * KB, rem)])
            plsc.subcore_barrier()

            def issue(bi, p):
                off = sid * PER_TILE + bi * KB
                pltpu.sync_copy(src_hbm.at[pl.ds(off, KB)], srcv.at[p])
                pltpu.sync_copy(dst_hbm.at[pl.ds(off, KB)], dstv.at[p])
                pltpu.async_copy(xl_hbm.at[srcv.at[p]], xlv.at[p], semg[p])
                pltpu.async_copy(xr_hbm.at[dstv.at[p]], xrv.at[p], semg[p])

            def wait_gathers(p):
                pltpu.make_async_copy(
                    xl_hbm.at[srcv.at[p]], xlv.at[p], semg[p]).wait()
                pltpu.make_async_copy(
                    xr_hbm.at[dstv.at[p]], xrv.at[p], semg[p]).wait()

            def consume(bi, p):
                off = sid * PER_TILE + bi * KB
                pv = jnp.full((16,), p, jnp.int32)

                def mask16(j, c2):
                    dv = dstv[p, pl.ds(j * 16, 16)]
                    gidx = off + j * 16 + lanes
                    inm = (dv >= lo) & (dv < lo + CH) & (gidx < E_REAL)
                    dlocv[pl.ds(j * 16, 16)] = jnp.where(inm, dv - lo, 0)
                    wtv[pl.ds(j * 16, 16)] = jnp.where(inm, 1.0, 0.0)
                    return c2

                lax.fori_loop(0, KB // 16, mask16, 0)

                def egroup(g, c3):
                    rows = lanes + g * 16
                    wt16 = wtv[pl.ds(g * 16, 16)]
                    for h in range(heads):
                        acc = zero16
                        avs = []
                        for c in range(outc):
                            o = h * outc + c
                            colv = jnp.full((16,), o, jnp.int32)
                            a = plsc.load_gather(xlv, [pv, rows, colv])
                            b = plsc.load_gather(xrv, [pv, rows, colv])
                            avs.append(a)
                            m = a + b
                            m = jnp.where(m > 0, m, m * 0.2)
                            acc = acc + m * att_s[o]
                        ex = jnp.exp(acc) * wt16
                        for c in range(outc):
                            o = h * outc + c
                            colv = jnp.full((16,), o, jnp.int32)
                            plsc.store_scatter(valsv, [rows, colv],
                                               avs[c] * ex)
                        plsc.store_scatter(
                            valsv, [rows, jnp.full((16,), d + h, jnp.int32)],
                            ex)
                    return c3

                lax.fori_loop(0, KB // 16, egroup, 0)
                pltpu.sync_copy(valsv, accsh.at[dlocv], add=True)

            if att_s is None:
                att_vregs = [attv[pl.ds(v * 16, 16)] for v in range(d // 16)]
                att_s = [att_vregs[o // 16][o % 16] for o in range(d)]

            issue(0, 0)

            def pair(s, carry):
                for p in range(2):
                    bi = 2 * s + p
                    wait_gathers(p)
                    issue(jnp.minimum(bi + 1, NB - 1), 1 - p)
                    consume(bi, p)
                return carry

            lax.fori_loop(0, NB // 2, pair, 0)
            wait_gathers(0)
            plsc.subcore_barrier()
            pltpu.sync_copy(accsh.at[pl.ds(base, RPT)],
                            acc_hbm.at[chunk, pl.ds(base, RPT)])
            plsc.subcore_barrier()

    return edge_kernel


_edge1 = _make_edge_kernel(4, 16)
_edge2 = _make_edge_kernel(1, 32)


def _proj1_body(x_ref, wl_ref, bl_ref, wr_ref, br_ref, xl_ref, xr_ref):
    xb = x_ref[...]
    dn = (((1,), (1,)), ((), ()))
    xl_ref[...] = lax.dot_general(
        xb, wl_ref[...], dn, preferred_element_type=jnp.float32) + bl_ref[...]
    xr_ref[...] = lax.dot_general(
        xb, wr_ref[...], dn, preferred_element_type=jnp.float32) + br_ref[...]


def _proj1(x, wl, bl, wr, br):
    n = x.shape[0]
    blk = 1000
    return pl.pallas_call(
        _proj1_body,
        grid=(n // blk,),
        in_specs=[
            pl.BlockSpec((blk, 5), lambda i: (i, 0)),
            pl.BlockSpec((64, 5), lambda i: (0, 0)),
            pl.BlockSpec((64,), lambda i: (0,)),
            pl.BlockSpec((64, 5), lambda i: (0, 0)),
            pl.BlockSpec((64,), lambda i: (0,)),
        ],
        out_specs=[
            pl.BlockSpec((blk, 64), lambda i: (i, 0)),
            pl.BlockSpec((blk, 64), lambda i: (i, 0)),
        ],
        out_shape=[
            jax.ShapeDtypeStruct((n, 64), jnp.float32),
            jax.ShapeDtypeStruct((n, 64), jnp.float32),
        ],
    )(x, wl, bl, wr, br)


def _finish1_body(acc_ref, b1_ref, g1_ref, be1_ref, wl2_ref, bl2_ref,
                  wr2_ref, br2_ref, xl2_ref, xr2_ref):
    blk = acc_ref[0]
    num = blk[:, :64]
    den4 = blk[:, 64:68]
    denf = jnp.reshape(
        jnp.broadcast_to(den4[:, :, None], (den4.shape[0], 4, 16)),
        (den4.shape[0], 64))
    h = num / (denf + 1e-16) + b1_ref[...]
    mu = jnp.mean(h, axis=-1, keepdims=True)
    var = jnp.mean((h - mu) ** 2, axis=-1, keepdims=True)
    h = (h - mu) * lax.rsqrt(var + 1e-5) * g1_ref[...] + be1_ref[...]
    h = jnp.where(h > 0, h, jnp.exp(h) - 1.0)
    dn = (((1,), (1,)), ((), ()))
    xl2_ref[...] = lax.dot_general(
        h, wl2_ref[...], dn, preferred_element_type=jnp.float32) + bl2_ref[...]
    xr2_ref[...] = lax.dot_general(
        h, wr2_ref[...], dn, preferred_element_type=jnp.float32) + br2_ref[...]


def _finish1(acc, b1, g1, be1, wl2, bl2, wr2, br2):
    blk = 1000
    nb = CH // blk
    return pl.pallas_call(
        _finish1_body,
        grid=(NCH, nb),
        in_specs=[
            pl.BlockSpec((1, blk, 80), lambda c, r: (c, r, 0)),
            pl.BlockSpec((64,), lambda c, r: (0,)),
            pl.BlockSpec((64,), lambda c, r: (0,)),
            pl.BlockSpec((64,), lambda c, r: (0,)),
            pl.BlockSpec((32, 64), lambda c, r: (0, 0)),
            pl.BlockSpec((32,), lambda c, r: (0,)),
            pl.BlockSpec((32, 64), lambda c, r: (0, 0)),
            pl.BlockSpec((32,), lambda c, r: (0,)),
        ],
        out_specs=[
            pl.BlockSpec((blk, 32), lambda c, r: (c * nb + r, 0)),
            pl.BlockSpec((blk, 32), lambda c, r: (c * nb + r, 0)),
        ],
        out_shape=[
            jax.ShapeDtypeStruct((NCH * CH, 32), jnp.float32),
            jax.ShapeDtypeStruct((NCH * CH, 32), jnp.float32),
        ],
    )(acc, b1, g1, be1, wl2, bl2, wr2, br2)


def _finish2_body(acc_ref, b2_ref, g2_ref, be2_ref, o_ref):
    blk = acc_ref[0]
    num = blk[:, :32]
    den = blk[:, 32:33]
    h = num / (jnp.broadcast_to(den, num.shape) + 1e-16) + b2_ref[...]
    mu = jnp.mean(h, axis=-1, keepdims=True)
    var = jnp.mean((h - mu) ** 2, axis=-1, keepdims=True)
    o_ref[...] = (h - mu) * lax.rsqrt(var + 1e-5) * g2_ref[...] + be2_ref[...]


def _finish2(acc, b2, g2, be2):
    blk = 1000
    nb = CH // blk
    return pl.pallas_call(
        _finish2_body,
        grid=(NCH, nb),
        in_specs=[
            pl.BlockSpec((1, blk, 48), lambda c, r: (c, r, 0)),
            pl.BlockSpec((32,), lambda c, r: (0,)),
            pl.BlockSpec((32,), lambda c, r: (0,)),
            pl.BlockSpec((32,), lambda c, r: (0,)),
        ],
        out_specs=pl.BlockSpec((blk, 32), lambda c, r: (c * nb + r, 0)),
        out_shape=jax.ShapeDtypeStruct((NCH * CH, 32), jnp.float32),
    )(acc, b2, g2, be2)


def kernel(x, edge_index, Wl1, bl1, Wr1, br1, att1, bias1, g1, beta1,
           Wl2, bl2, Wr2, br2, att2, bias2, g2, beta2):
    n = x.shape[0]
    e = edge_index.shape[1]
    loop = jnp.arange(n, dtype=edge_index.dtype)
    pad = E_PAD - (e + n)
    src = jnp.concatenate(
        [edge_index[0], loop, jnp.zeros((pad,), edge_index.dtype)])
    dst = jnp.concatenate(
        [edge_index[1], loop, jnp.zeros((pad,), edge_index.dtype)])

    xl1, xr1 = _proj1(x, Wl1, bl1, Wr1, br1)
    acc1 = _edge1(src, dst, xl1, xr1, att1.reshape(64))
    xl2, xr2 = _finish1(acc1, bias1, g1, beta1, Wl2, bl2, Wr2, br2)
    acc2 = _edge2(src, dst, xl2, xr2, att2.reshape(32))
    return _finish2(acc2, bias2, g2, beta2)[:n]
